# speedup vs baseline: 7.2801x; 7.2801x over previous
import functools

import jax
import jax.numpy as jnp
from jax import lax
from jax.experimental import pallas as pl
from jax.experimental.pallas import tpu as pltpu
from jax.experimental.pallas import tpu_sc as plsc

DIM = 128
EPS = 1e-07

_NC = 2
_NS = 16
_NW = _NC * _NS
_CHUNK = 128


def _sc_gather(ids_flat, word_table):
    n_tok = ids_flat.shape[0]
    assert n_tok % (_NW * _CHUNK) == 0
    per_w = n_tok // _NW
    n_chunks = per_w // _CHUNK

    mesh = plsc.VectorSubcoreMesh(core_axis_name="c", subcore_axis_name="s")

    @functools.partial(
        pl.kernel,
        out_type=jax.ShapeDtypeStruct((n_tok, DIM), jnp.float32),
        mesh=mesh,
        scratch_types=[
            pltpu.VMEM((_CHUNK,), jnp.int32),
            pltpu.VMEM((_CHUNK, DIM), jnp.float32),
            pltpu.SemaphoreType.DMA,
        ],
    )
    def gather_kernel(ids_hbm, table_hbm, out_hbm, idx_v, rows_v, sem):
        wid = lax.axis_index("s") * _NC + lax.axis_index("c")
        base = wid * per_w

        def body(c, carry):
            off = base + c * _CHUNK
            pltpu.sync_copy(ids_hbm.at[pl.ds(off, _CHUNK)], idx_v)
            pltpu.async_copy(table_hbm.at[idx_v], rows_v, sem).wait()
            pltpu.sync_copy(rows_v, out_hbm.at[pl.ds(off, _CHUNK)])
            return carry

        lax.fori_loop(0, n_chunks, body, 0)

    return gather_kernel(ids_flat, word_table)


def _tc_body(g_ref, tt_ref, pos_ref, ty_ref, gb_ref, emb_ref, pos_out_ref):
    x = g_ref[...]
    pos = pos_ref[...]
    ty = ty_ref[...]
    gb = gb_ref[...]
    ttf = tt_ref[...]

    x = x + pos[None]
    t0 = ty[0][None, None, :]
    dt = (ty[1] - ty[0])[None, None, :]
    x = x + t0 + ttf[:, :, None] * dt

    mu = jnp.mean(x, axis=-1, keepdims=True)
    xc = x - mu
    var = jnp.mean(xc * xc, axis=-1, keepdims=True)
    y = xc * lax.rsqrt(var + EPS)
    y = y * gb[0][None, None, :] + gb[1][None, None, :]

    emb_ref[...] = y
    pos_out_ref[...] = jnp.broadcast_to(pos[None], x.shape)


def kernel(input_ids, token_type_ids, word_table, pos_table, type_table, gamma, beta):
    b, s = input_ids.shape
    ids_flat = input_ids.reshape(-1).astype(jnp.int32)
    gathered = _sc_gather(ids_flat, word_table).reshape(b, s, DIM)

    ttf = token_type_ids.astype(jnp.float32)
    pos_s = pos_table[:s]
    gb = jnp.stack([gamma, beta])

    br = 8
    grid = (b // br,)
    out_shape = [
        jax.ShapeDtypeStruct((b, s, DIM), jnp.float32),
        jax.ShapeDtypeStruct((b, s, DIM), jnp.float32),
    ]
    emb, pos_out = pl.pallas_call(
        _tc_body,
        grid=grid,
        in_specs=[
            pl.BlockSpec((br, s, DIM), lambda i: (i, 0, 0)),
            pl.BlockSpec((br, s), lambda i: (i, 0)),
            pl.BlockSpec((s, DIM), lambda i: (0, 0)),
            pl.BlockSpec((2, DIM), lambda i: (0, 0)),
            pl.BlockSpec((2, DIM), lambda i: (0, 0)),
        ],
        out_specs=[
            pl.BlockSpec((br, s, DIM), lambda i: (i, 0, 0)),
            pl.BlockSpec((br, s, DIM), lambda i: (i, 0, 0)),
        ],
        out_shape=out_shape,
    )(gathered, ttf, pos_s, type_table, gb)

    return emb, pos_out
